# Initial kernel scaffold; baseline (speedup 1.0000x reference)
import jax
import jax.numpy as jnp
from jax.experimental import pallas as pl


def kernel(inputs, W_head, b_head, g_head, bt_head, W_b0, b_b0, g_b0, bt_b0, W_b1, b_b1, g_b1, bt_b1, W_fu, b_fu, g_fu, bt_fu, W_p0, b_p0, g_p0, bt_p0, W_p1, b_p1, g_p1, bt_p1, W_p2, b_p2):
    raise NotImplementedError("write your pallas kernel here")


# speedup vs baseline: 10.9119x; 10.9119x over previous
import functools

import numpy as np
import jax
import jax.numpy as jnp
from jax import lax
from jax.experimental import pallas as pl
from jax.experimental.pallas import tpu as pltpu
from jax.experimental.pallas import tpu_sc as plsc

_B, _N, _DIM, _CIN, _K, _NC = 4, 2048, 3, 9, 16, 13
_SQ = float(np.sqrt(np.float32(1.0 + 1e-5)))
_RB = 256
_NBLK = _N // _RB
_TOT = _B * _N
_NWORK = 32
_NEDGE = _B * _K * _N
_HI = lax.Precision.HIGHEST
_BF = jnp.bfloat16


def _topk_rows(score, k, stride, boff):
    colid = lax.broadcasted_iota(jnp.int32, score.shape, 1)
    neg = jnp.float32(-jnp.inf)
    rows = []
    for t in range(k):
        it = jnp.argmax(score, axis=1).astype(jnp.int32)
        if t % stride == 0:
            rows.append(it[None, :] + boff)
        score = jnp.where(colid == it[:, None], neg, score)
    return jnp.concatenate(rows, axis=0)


def _score_block(rows, full):
    nt = (((1,), (1,)), ((), ()))
    g = lax.dot_general(rows.astype(_BF), full.astype(_BF), nt,
                        preferred_element_type=jnp.float32)
    x2r = jnp.sum(rows * rows, axis=1, keepdims=True)
    ones = jnp.ones((1, full.shape[1]), jnp.float32)
    x2c = lax.dot_general(ones, full * full, nt, precision=_HI,
                          preferred_element_type=jnp.float32)
    return 2.0 * g - x2r - x2c


def _mmb(x, w):
    return lax.dot_general(x.astype(_BF), w.astype(_BF),
                           (((1,), (0,)), ((), ())),
                           preferred_element_type=jnp.float32)


def _knn_call(x, csel, k, stride):
    cx = x.shape[2]

    def body(x_ref, idx_ref):
        b, i = pl.program_id(0), pl.program_id(1)
        off = pl.multiple_of(i * _RB, _RB)
        f = x_ref[0]
        rows = x_ref[0, pl.ds(off, _RB)]
        if csel != cx:
            f = f[:, :csel]
            rows = rows[:, :csel]
        score = _score_block(rows, f)
        idx_ref[0] = _topk_rows(score, k, stride, b * _N)

    ko = k // stride
    return pl.pallas_call(
        body,
        grid=(_B, _NBLK),
        in_specs=[pl.BlockSpec((1, _N, cx), lambda b, i: (b, 0, 0))],
        out_specs=[pl.BlockSpec((1, ko, _RB), lambda b, i: (b, 0, i))],
        out_shape=[jax.ShapeDtypeStruct((_B, ko, _N), jnp.int32)],
    )(x)[0]


def _gather_rows(table, idxflat):
    rpw = _NEDGE // _NWORK
    mesh = plsc.VectorSubcoreMesh(core_axis_name="c", subcore_axis_name="s")

    @functools.partial(
        pl.kernel, mesh=mesh,
        out_type=jax.ShapeDtypeStruct((_NEDGE, 128), jnp.float32),
        scratch_types=[
            pltpu.VMEM((128,), jnp.int32),
            pltpu.VMEM((128, 128), jnp.float32),
            pltpu.SemaphoreType.DMA,
        ],
    )
    def k(tab_hbm, idx_hbm, out_hbm, idx_v, rows_v, sem):
        wid = lax.axis_index("c") * 16 + lax.axis_index("s")
        base = wid * rpw
        nround = rpw // 128

        def round_body(g, carry):
            pltpu.sync_copy(idx_hbm.at[pl.ds(base + g * 128, 128)], idx_v)
            pltpu.async_copy(tab_hbm.at[idx_v], rows_v, sem).wait()
            pltpu.sync_copy(rows_v, out_hbm.at[pl.ds(base + g * 128, 128)])
            return carry

        lax.fori_loop(0, nround, round_body, 0)

    return k(table, idxflat)


def _edge_call(x, gath, w2c, bb, gg, bt, cx, out_w, pass_through):
    cxl = x.shape[2]
    g4 = gath.reshape(_B, _K, _N, 128)

    def body(x_ref, g_ref, w_ref, bb_ref, gg_ref, bt_ref, o_ref):
        xi = x_ref[0]
        xb = xi.astype(_BF)
        wb = w_ref[...].astype(_BF)
        m = None
        for kk in range(_K):
            xj = g_ref[0, kk][:, :cxl]
            e = (xj - xi).astype(_BF)
            z = lax.dot_general(jnp.concatenate([xb, e], axis=1), wb,
                                (((1,), (0,)), ((), ())),
                                preferred_element_type=jnp.float32)
            z = gg_ref[...] * ((z + bb_ref[...]) / _SQ) + bt_ref[...]
            m = z if m is None else jnp.maximum(m, z)
        d = jnp.maximum(m, 0.0)
        if pass_through:
            o_ref[0, :, :64] = xi[:, :64]
            o_ref[0, :, 64:128] = d
        else:
            o_ref[0, :, :64] = d

    wspec = lambda s: pl.BlockSpec(s, lambda b, i: (0, 0))
    return pl.pallas_call(
        body,
        grid=(_B, _NBLK),
        in_specs=[
            pl.BlockSpec((1, _RB, cxl), lambda b, i: (b, i, 0)),
            pl.BlockSpec((1, _K, _RB, 128), lambda b, i: (b, 0, i, 0)),
            wspec((2 * cxl, 64)), wspec((1, 64)), wspec((1, 64)),
            wspec((1, 64)),
        ],
        out_specs=[pl.BlockSpec((1, _RB, out_w), lambda b, i: (b, i, 0))],
        out_shape=[jax.ShapeDtypeStruct((_B, _N, out_w), jnp.float32)],
    )(x, g4, w2c, bb, gg, bt)[0]


def _tail_body(f1_ref, d1_ref, wfu_ref, bfu_ref, gfu_ref, btfu_ref,
               wp0t_ref, wp0b_ref, bp0_ref, gp0_ref, btp0_ref,
               wp1_ref, bp1_ref, gp1_ref, btp1_ref,
               wp2_ref, bp2_ref, o_ref):
    f1 = f1_ref[0]
    d1 = d1_ref[0][:, :64]
    feats = jnp.concatenate([f1[:, :64], f1, f1, d1], axis=1)
    gm = jnp.zeros((1, 1024), jnp.float32)
    for blk in range(_NBLK):
        fr = feats[blk * _RB:(blk + 1) * _RB]
        z = _mmb(fr, wfu_ref[...]) + bfu_ref[...]
        z = gfu_ref[...] * (z / _SQ) + btfu_ref[...]
        fu = jnp.maximum(z, 0.0)
        gm = jnp.maximum(gm, jnp.max(fu, axis=0, keepdims=True))
    r = _mmb(gm, wp0t_ref[...])
    for blk in range(_NBLK):
        fr = feats[blk * _RB:(blk + 1) * _RB]
        z = _mmb(fr, wp0b_ref[...]) + r + bp0_ref[...]
        z = gp0_ref[...] * (z / _SQ) + btp0_ref[...]
        h0 = jnp.maximum(z, 0.0)
        z = _mmb(h0, wp1_ref[...]) + bp1_ref[...]
        z = gp1_ref[...] * (z / _SQ) + btp1_ref[...]
        h1 = jnp.maximum(z, 0.0)
        o_ref[0, pl.ds(blk * _RB, _RB)] = _mmb(h1, wp2_ref[...]) + bp2_ref[...]


def _tail_call(f1, d1, wfu, bfu, gfu, btfu, wp0t, wp0b, bp0, gp0, btp0,
               wp1, bp1, gp1, btp1, wp2, bp2):
    full = lambda c: pl.BlockSpec((1, _N, c), lambda b: (b, 0, 0))
    wspec = lambda s: pl.BlockSpec(s, lambda b: (0, 0))
    return pl.pallas_call(
        _tail_body,
        grid=(_B,),
        in_specs=[full(128), full(128),
                  wspec((384, 1024)), wspec((1, 1024)), wspec((1, 1024)),
                  wspec((1, 1024)),
                  wspec((1024, 512)), wspec((384, 512)), wspec((1, 512)),
                  wspec((1, 512)), wspec((1, 512)),
                  wspec((512, 256)), wspec((1, 256)), wspec((1, 256)),
                  wspec((1, 256)),
                  wspec((256, 128)), wspec((1, 128))],
        out_specs=[full(128)],
        out_shape=[jax.ShapeDtypeStruct((_B, _N, 128), jnp.float32)],
    )(f1, d1, wfu, bfu, gfu, btfu, wp0t, wp0b, bp0, gp0, btp0,
      wp1, bp1, gp1, btp1, wp2, bp2)[0]


def kernel(inputs, W_head, b_head, g_head, bt_head, W_b0, b_b0, g_b0, bt_b0,
           W_b1, b_b1, g_b1, bt_b1, W_fu, b_fu, g_fu, bt_fu,
           W_p0, b_p0, g_p0, bt_p0, W_p1, b_p1, g_p1, bt_p1, W_p2, b_p2):
    f32 = jnp.float32
    x0 = jnp.transpose(inputs[..., 0], (0, 2, 1)).astype(f32)
    x0p = jnp.pad(x0, ((0, 0), (0, 0), (0, 16 - _CIN)))
    posp = jnp.pad(x0[:, :, :_DIM], ((0, 0), (0, 0), (0, 16 - _DIM)))
    x0w = jnp.pad(x0, ((0, 0), (0, 0), (0, 128 - _CIN)))

    whead = jnp.zeros((32, 64), f32)
    whead = whead.at[0:_CIN].set(W_head[0:_CIN])
    whead = whead.at[16:16 + _CIN].set(W_head[_CIN:])
    r1 = lambda v: v[None, :]

    idx0 = _knn_call(posp, 16, _K, 1)
    g0 = _gather_rows(x0w.reshape(_TOT, 128), idx0.reshape(-1))
    f0w = _edge_call(x0p, g0, whead, r1(b_head), r1(g_head), r1(bt_head),
                     _CIN, 128, False)

    idx1 = _knn_call(f0w, 64, _K, 1)
    g1 = _gather_rows(f0w.reshape(_TOT, 128), idx1.reshape(-1))
    f0c = f0w[:, :, :64]
    f1w = _edge_call(f0c, g1, W_b0, r1(b_b0), r1(g_b0), r1(bt_b0),
                     64, 128, True)

    idx2 = _knn_call(f1w, 128, 2 * _K, 2)
    g2 = _gather_rows(f1w.reshape(_TOT, 128), idx2.reshape(-1))
    d1w = _edge_call(f1w, g2, W_b1, r1(b_b1), r1(g_b1), r1(bt_b1),
                     128, 128, False)

    wp0t, wp0b = W_p0[:1024], W_p0[1024:]
    wp2 = jnp.pad(W_p2, ((0, 0), (0, 128 - _NC)))
    bp2 = jnp.pad(b_p2, ((0, 128 - _NC)))[None, :]
    o = _tail_call(f1w, d1w, W_fu, r1(b_fu), r1(g_fu), r1(bt_fu),
                   wp0t, wp0b, r1(b_p0), r1(g_p0), r1(bt_p0),
                   W_p1, r1(b_p1), r1(g_p1), r1(bt_p1), wp2, bp2)
    return jnp.transpose(o[:, :, :_NC], (0, 2, 1))


# speedup vs baseline: 11.6763x; 1.0701x over previous
import functools

import numpy as np
import jax
import jax.numpy as jnp
from jax import lax
from jax.experimental import pallas as pl
from jax.experimental.pallas import tpu as pltpu
from jax.experimental.pallas import tpu_sc as plsc

_B, _N, _DIM, _CIN, _K, _NC = 4, 2048, 3, 9, 16, 13
_SQ = float(np.sqrt(np.float32(1.0 + 1e-5)))
_RB = 256
_NBLK = _N // _RB
_TOT = _B * _N
_NWORK = 32
_NEDGE = _B * _K * _N
_HI = lax.Precision.HIGHEST
_BF = jnp.bfloat16


def _topk_rows(score, k, stride, boff):
    colid = lax.broadcasted_iota(jnp.int32, score.shape, 1)
    neg = jnp.float32(-jnp.inf)
    rows = []
    for t in range(k):
        it = jnp.argmax(score, axis=1).astype(jnp.int32)
        if t % stride == 0:
            rows.append(it[None, :] + boff)
        score = jnp.where(colid == it[:, None], neg, score)
    return jnp.concatenate(rows, axis=0)


def _score_block(rows, full):
    nt = (((1,), (1,)), ((), ()))
    g = lax.dot_general(rows.astype(_BF), full.astype(_BF), nt,
                        preferred_element_type=jnp.float32)
    x2r = jnp.sum(rows * rows, axis=1, keepdims=True)
    ones = jnp.ones((1, full.shape[1]), jnp.float32)
    x2c = lax.dot_general(ones, full * full, nt, precision=_HI,
                          preferred_element_type=jnp.float32)
    return 2.0 * g - x2r - x2c


def _mmb(x, w):
    return lax.dot_general(x.astype(_BF), w.astype(_BF),
                           (((1,), (0,)), ((), ())),
                           preferred_element_type=jnp.float32)


def _knn_call(x, csel, k, stride):
    cx = x.shape[2]

    def body(x_ref, idx_ref):
        b, i = pl.program_id(0), pl.program_id(1)
        off = pl.multiple_of(i * _RB, _RB)
        f = x_ref[0]
        rows = x_ref[0, pl.ds(off, _RB)]
        if csel != cx:
            f = f[:, :csel]
            rows = rows[:, :csel]
        score = _score_block(rows, f)
        idx_ref[0] = _topk_rows(score, k, stride, b * _N)

    ko = k // stride
    return pl.pallas_call(
        body,
        grid=(_B, _NBLK),
        in_specs=[pl.BlockSpec((1, _N, cx), lambda b, i: (b, 0, 0))],
        out_specs=[pl.BlockSpec((1, ko, _RB), lambda b, i: (b, 0, i))],
        out_shape=[jax.ShapeDtypeStruct((_B, ko, _N), jnp.int32)],
    )(x)[0]


def _gather_rows(table, idxflat):
    rpw = _NEDGE // _NWORK
    mesh = plsc.VectorSubcoreMesh(core_axis_name="c", subcore_axis_name="s")

    @functools.partial(
        pl.kernel, mesh=mesh,
        out_type=jax.ShapeDtypeStruct((_NEDGE, 128), jnp.float32),
        scratch_types=[
            pltpu.VMEM((128,), jnp.int32),
            pltpu.VMEM((128, 128), jnp.float32),
            pltpu.VMEM((128,), jnp.int32),
            pltpu.VMEM((128, 128), jnp.float32),
            pltpu.SemaphoreType.DMA,
            pltpu.SemaphoreType.DMA,
        ],
    )
    def k(tab_hbm, idx_hbm, out_hbm, i0, r0, i1, r1, s0, s1):
        wid = lax.axis_index("c") * 16 + lax.axis_index("s")
        base = wid * rpw
        nround = rpw // 128

        def start(g, iv, rv, sem):
            pltpu.sync_copy(idx_hbm.at[pl.ds(base + g * 128, 128)], iv)
            pltpu.async_copy(tab_hbm.at[iv], rv, sem)

        def fin(g, iv, rv, sem):
            pltpu.make_async_copy(tab_hbm.at[iv], rv, sem).wait()
            pltpu.sync_copy(rv, out_hbm.at[pl.ds(base + g * 128, 128)])

        start(0, i0, r0, s0)

        def pair_body(h, carry):
            g = h * 2
            start(g + 1, i1, r1, s1)
            fin(g, i0, r0, s0)

            @pl.when(g + 2 < nround)
            def _():
                start(g + 2, i0, r0, s0)

            fin(g + 1, i1, r1, s1)
            return carry

        lax.fori_loop(0, nround // 2, pair_body, 0)

    return k(table, idxflat)


def _edge_call(x, gath, w2c, bb, gg, bt, cx, out_w, pass_through):
    cxl = x.shape[2]
    g4 = gath.reshape(_B, _K, _N, 128)

    def body(x_ref, g_ref, w_ref, bb_ref, gg_ref, bt_ref, o_ref):
        xi = x_ref[0]
        xb = xi.astype(_BF)
        wb = w_ref[...].astype(_BF)
        m = None
        for kk in range(_K):
            xj = g_ref[0, kk][:, :cxl]
            e = (xj - xi).astype(_BF)
            z = lax.dot_general(jnp.concatenate([xb, e], axis=1), wb,
                                (((1,), (0,)), ((), ())),
                                preferred_element_type=jnp.float32)
            z = gg_ref[...] * ((z + bb_ref[...]) / _SQ) + bt_ref[...]
            m = z if m is None else jnp.maximum(m, z)
        d = jnp.maximum(m, 0.0)
        if pass_through:
            o_ref[0, :, :64] = xi[:, :64]
            o_ref[0, :, 64:128] = d
        else:
            o_ref[0, :, :64] = d

    wspec = lambda s: pl.BlockSpec(s, lambda b, i: (0, 0))
    return pl.pallas_call(
        body,
        grid=(_B, _NBLK),
        in_specs=[
            pl.BlockSpec((1, _RB, cxl), lambda b, i: (b, i, 0)),
            pl.BlockSpec((1, _K, _RB, 128), lambda b, i: (b, 0, i, 0)),
            wspec((2 * cxl, 64)), wspec((1, 64)), wspec((1, 64)),
            wspec((1, 64)),
        ],
        out_specs=[pl.BlockSpec((1, _RB, out_w), lambda b, i: (b, i, 0))],
        out_shape=[jax.ShapeDtypeStruct((_B, _N, out_w), jnp.float32)],
    )(x, g4, w2c, bb, gg, bt)[0]


def _tail_body(f1_ref, d1_ref, wfu_ref, bfu_ref, gfu_ref, btfu_ref,
               wp0t_ref, wp0b_ref, bp0_ref, gp0_ref, btp0_ref,
               wp1_ref, bp1_ref, gp1_ref, btp1_ref,
               wp2_ref, bp2_ref, o_ref):
    f1 = f1_ref[0]
    d1 = d1_ref[0][:, :64]
    feats = jnp.concatenate([f1[:, :64], f1, f1, d1], axis=1)
    gm = jnp.zeros((1, 1024), jnp.float32)
    for blk in range(_NBLK):
        fr = feats[blk * _RB:(blk + 1) * _RB]
        z = _mmb(fr, wfu_ref[...]) + bfu_ref[...]
        z = gfu_ref[...] * (z / _SQ) + btfu_ref[...]
        fu = jnp.maximum(z, 0.0)
        gm = jnp.maximum(gm, jnp.max(fu, axis=0, keepdims=True))
    r = _mmb(gm, wp0t_ref[...])
    for blk in range(_NBLK):
        fr = feats[blk * _RB:(blk + 1) * _RB]
        z = _mmb(fr, wp0b_ref[...]) + r + bp0_ref[...]
        z = gp0_ref[...] * (z / _SQ) + btp0_ref[...]
        h0 = jnp.maximum(z, 0.0)
        z = _mmb(h0, wp1_ref[...]) + bp1_ref[...]
        z = gp1_ref[...] * (z / _SQ) + btp1_ref[...]
        h1 = jnp.maximum(z, 0.0)
        o_ref[0, pl.ds(blk * _RB, _RB)] = _mmb(h1, wp2_ref[...]) + bp2_ref[...]


def _tail_call(f1, d1, wfu, bfu, gfu, btfu, wp0t, wp0b, bp0, gp0, btp0,
               wp1, bp1, gp1, btp1, wp2, bp2):
    full = lambda c: pl.BlockSpec((1, _N, c), lambda b: (b, 0, 0))
    wspec = lambda s: pl.BlockSpec(s, lambda b: (0, 0))
    return pl.pallas_call(
        _tail_body,
        grid=(_B,),
        in_specs=[full(128), full(128),
                  wspec((384, 1024)), wspec((1, 1024)), wspec((1, 1024)),
                  wspec((1, 1024)),
                  wspec((1024, 512)), wspec((384, 512)), wspec((1, 512)),
                  wspec((1, 512)), wspec((1, 512)),
                  wspec((512, 256)), wspec((1, 256)), wspec((1, 256)),
                  wspec((1, 256)),
                  wspec((256, 128)), wspec((1, 128))],
        out_specs=[full(128)],
        out_shape=[jax.ShapeDtypeStruct((_B, _N, 128), jnp.float32)],
    )(f1, d1, wfu, bfu, gfu, btfu, wp0t, wp0b, bp0, gp0, btp0,
      wp1, bp1, gp1, btp1, wp2, bp2)[0]


def kernel(inputs, W_head, b_head, g_head, bt_head, W_b0, b_b0, g_b0, bt_b0,
           W_b1, b_b1, g_b1, bt_b1, W_fu, b_fu, g_fu, bt_fu,
           W_p0, b_p0, g_p0, bt_p0, W_p1, b_p1, g_p1, bt_p1, W_p2, b_p2):
    f32 = jnp.float32
    x0 = jnp.transpose(inputs[..., 0], (0, 2, 1)).astype(f32)
    x0p = jnp.pad(x0, ((0, 0), (0, 0), (0, 16 - _CIN)))
    posp = jnp.pad(x0[:, :, :_DIM], ((0, 0), (0, 0), (0, 16 - _DIM)))
    x0w = jnp.pad(x0, ((0, 0), (0, 0), (0, 128 - _CIN)))

    whead = jnp.zeros((32, 64), f32)
    whead = whead.at[0:_CIN].set(W_head[0:_CIN])
    whead = whead.at[16:16 + _CIN].set(W_head[_CIN:])
    r1 = lambda v: v[None, :]

    idx0 = _knn_call(posp, 16, _K, 1)
    g0 = _gather_rows(x0w.reshape(_TOT, 128), idx0.reshape(-1))
    f0w = _edge_call(x0p, g0, whead, r1(b_head), r1(g_head), r1(bt_head),
                     _CIN, 128, False)

    idx1 = _knn_call(f0w, 64, _K, 1)
    g1 = _gather_rows(f0w.reshape(_TOT, 128), idx1.reshape(-1))
    f0c = f0w[:, :, :64]
    f1w = _edge_call(f0c, g1, W_b0, r1(b_b0), r1(g_b0), r1(bt_b0),
                     64, 128, True)

    idx2 = _knn_call(f1w, 128, 2 * _K, 2)
    g2 = _gather_rows(f1w.reshape(_TOT, 128), idx2.reshape(-1))
    d1w = _edge_call(f1w, g2, W_b1, r1(b_b1), r1(g_b1), r1(bt_b1),
                     128, 128, False)

    wp0t, wp0b = W_p0[:1024], W_p0[1024:]
    wp2 = jnp.pad(W_p2, ((0, 0), (0, 128 - _NC)))
    bp2 = jnp.pad(b_p2, ((0, 128 - _NC)))[None, :]
    o = _tail_call(f1w, d1w, W_fu, r1(b_fu), r1(g_fu), r1(bt_fu),
                   wp0t, wp0b, r1(b_p0), r1(g_p0), r1(bt_p0),
                   W_p1, r1(b_p1), r1(g_p1), r1(bt_p1), wp2, bp2)
    return jnp.transpose(o[:, :, :_NC], (0, 2, 1))
